# Initial kernel scaffold; baseline (speedup 1.0000x reference)
import jax
import jax.numpy as jnp
from jax.experimental import pallas as pl


def kernel(x, edge_index, W1, b1, W2, b2):
    raise NotImplementedError("write your pallas kernel here")


# speedup vs baseline: 23.3834x; 23.3834x over previous
import functools

import jax
import jax.numpy as jnp
from jax import lax
from jax.experimental import pallas as pl
from jax.experimental.pallas import tpu as pltpu
from jax.experimental.pallas import tpu_sc as plsc

_N = 10000
_E = 320000
_D = 128
_D2 = 16
_NP = 10240
_NC, _NS = 2, 16
_NW = _NC * _NS
_EPT = _E // _NW
_CH = 80
_NCH = _EPT // _CH
_EPS = _E // _NS

_mesh = plsc.VectorSubcoreMesh(core_axis_name="c", subcore_axis_name="s")


def _deg_body(dst_hbm, deg_out, dst_v, hist_v, blk_v, acc_v, spm):
    cid = lax.axis_index("c")
    sid = lax.axis_index("s")
    zeros16 = jnp.zeros((16,), jnp.float32)
    ones16 = jnp.ones((16,), jnp.float32)

    @pl.when(cid == 0)
    def _():
        def zero(i, _):
            hist_v[pl.ds(i * 16, 16)] = zeros16
            return 0
        lax.fori_loop(0, _NP // 16, zero, 0)

        pltpu.sync_copy(dst_hbm.at[pl.ds(sid * _EPS, _EPS)], dst_v)

        def count(i, _):
            idx = dst_v[pl.ds(i * 16, 16)]
            plsc.addupdate_scatter(hist_v, [idx], ones16)
            return 0
        lax.fori_loop(0, _EPS // 16, count, 0)

        pltpu.sync_copy(hist_v, spm.at[sid])

    plsc.subcore_barrier()

    @pl.when(cid == 0)
    def _():
        pltpu.sync_copy(spm.at[:, pl.ds(sid * 640, 640)], blk_v)

        def reduce(j, _):
            s = blk_v[0, pl.ds(j * 16, 16)]
            for r in range(1, _NS):
                s = s + blk_v[r, pl.ds(j * 16, 16)]
            acc_v[pl.ds(j * 16, 16)] = s
            return 0
        lax.fori_loop(0, 640 // 16, reduce, 0)

        pltpu.sync_copy(acc_v, deg_out.at[pl.ds(sid * 640, 640)])


_sc_params = pltpu.CompilerParams(needs_layout_passes=False,
                                  use_tc_tiling_on_sc=False)

_deg_call = functools.partial(
    pl.kernel,
    out_type=jax.ShapeDtypeStruct((_NP,), jnp.float32),
    mesh=_mesh,
    compiler_params=_sc_params,
    scratch_types=[
        pltpu.VMEM((_EPS,), jnp.int32),
        pltpu.VMEM((_NP,), jnp.float32),
        pltpu.VMEM((_NS, 640), jnp.float32),
        pltpu.VMEM((640,), jnp.float32),
        pltpu.VMEM_SHARED((_NS, _NP), jnp.float32),
    ],
)(_deg_body)


def _agg_body(y_hbm, src_hbm, dst_hbm, zero_hbm, out_hbm,
              src_v, dst_v, rows_v, spm, sem):
    cid = lax.axis_index("c")
    sid = lax.axis_index("s")
    wid = sid * _NC + cid

    pltpu.sync_copy(zero_hbm.at[pl.ds(sid * 640, 640)],
                    spm.at[pl.ds(sid * 640, 640)])
    pltpu.sync_copy(src_hbm.at[wid], src_v)
    pltpu.sync_copy(dst_hbm.at[wid], dst_v)
    plsc.subcore_barrier()

    def chunk(j, _):
        pltpu.async_copy(y_hbm.at[src_v.at[j]], rows_v, sem).wait()
        pltpu.sync_copy(rows_v, spm.at[dst_v.at[j]], add=True)
        return 0
    lax.fori_loop(0, _NCH, chunk, 0)

    plsc.subcore_barrier()
    pltpu.sync_copy(spm.at[pl.ds(sid * 640, 640)],
                    out_hbm.at[cid, pl.ds(sid * 640, 640)])


def _make_agg(width):
    return functools.partial(
        pl.kernel,
        out_type=jax.ShapeDtypeStruct((_NC, _NP, width), jnp.float32),
        mesh=_mesh,
        compiler_params=_sc_params,
        scratch_types=[
            pltpu.VMEM((_NCH, _CH), jnp.int32),
            pltpu.VMEM((_NCH, _CH), jnp.int32),
            pltpu.VMEM((_CH, width), jnp.float32),
            pltpu.VMEM_SHARED((_NP, width), jnp.float32),
            pltpu.SemaphoreType.DMA,
        ],
    )(_agg_body)


_agg_call_d = _make_agg(_D)
_agg_call_2 = _make_agg(_D2)


_BR = 1000


def _y1_body(deg_ref, x_ref, w_ref, o_ref):
    dinv = lax.rsqrt(deg_ref[...] + 1.0)
    o_ref[...] = dinv * jnp.dot(x_ref[...], w_ref[...],
                                preferred_element_type=jnp.float32)


def _y1_call(degc, x, w1):
    return pl.pallas_call(
        _y1_body,
        grid=(_N // _BR,),
        in_specs=[
            pl.BlockSpec((_BR, 1), lambda i: (i, 0)),
            pl.BlockSpec((_BR, _D), lambda i: (i, 0)),
            pl.BlockSpec((_D, _D), lambda i: (0, 0)),
        ],
        out_specs=pl.BlockSpec((_BR, _D), lambda i: (i, 0)),
        out_shape=jax.ShapeDtypeStruct((_N, _D), jnp.float32),
    )(degc, x, w1)


def _mid_body(deg_ref, agg_ref, y1_ref, b1_ref, w2_ref, o_ref):
    dinv = lax.rsqrt(deg_ref[...] + 1.0)
    h = dinv * (agg_ref[0] + agg_ref[1] + y1_ref[...]) + b1_ref[...]
    h = jnp.maximum(h, 0.0)
    y2 = dinv * jnp.dot(h, w2_ref[...], preferred_element_type=jnp.float32)
    o_ref[...] = y2[:, :_D2]


def _mid_call(degc, agg1, y1, b1r, w2p):
    return pl.pallas_call(
        _mid_body,
        grid=(_N // _BR,),
        in_specs=[
            pl.BlockSpec((_BR, 1), lambda i: (i, 0)),
            pl.BlockSpec((_NC, _BR, _D), lambda i: (0, i, 0)),
            pl.BlockSpec((_BR, _D), lambda i: (i, 0)),
            pl.BlockSpec((1, _D), lambda i: (0, 0)),
            pl.BlockSpec((_D, _D), lambda i: (0, 0)),
        ],
        out_specs=pl.BlockSpec((_BR, _D2), lambda i: (i, 0)),
        out_shape=jax.ShapeDtypeStruct((_N, _D2), jnp.float32),
    )(degc, agg1, y1, b1r, w2p)


def _fin_body(deg_ref, agg_ref, y2_ref, b2_ref, o_ref):
    dinv = lax.rsqrt(deg_ref[...] + 1.0)
    o_ref[...] = dinv * (agg_ref[0] + agg_ref[1] + y2_ref[...]) + b2_ref[...]


def _fin_call(degc, agg2, y2, b2r):
    return pl.pallas_call(
        _fin_body,
        grid=(_N // _BR,),
        in_specs=[
            pl.BlockSpec((_BR, 1), lambda i: (i, 0)),
            pl.BlockSpec((_NC, _BR, _D2), lambda i: (0, i, 0)),
            pl.BlockSpec((_BR, _D2), lambda i: (i, 0)),
            pl.BlockSpec((1, _D2), lambda i: (0, 0)),
        ],
        out_specs=pl.BlockSpec((_BR, _D2), lambda i: (i, 0)),
        out_shape=jax.ShapeDtypeStruct((_N, _D2), jnp.float32),
    )(degc, agg2, y2, b2r)


def kernel(x, edge_index, W1, b1, W2, b2):
    x = x.astype(jnp.float32)
    src3 = edge_index[0].reshape(_NW, _NCH, _CH)
    dst3 = edge_index[1].reshape(_NW, _NCH, _CH)

    deg = _deg_call(edge_index[1])
    degc = deg[:_N].reshape(_N, 1)

    y1 = _y1_call(degc, x, W1)
    zeros1 = jnp.zeros((_NP, _D), jnp.float32)
    agg1 = _agg_call_d(y1, src3, dst3, zeros1)

    w2p = jnp.pad(W2, ((0, 0), (0, _D - W2.shape[1])))
    y2 = _mid_call(degc, agg1, y1, b1.reshape(1, _D), w2p)

    zeros2 = jnp.zeros((_NP, _D2), jnp.float32)
    agg2 = _agg_call_2(y2, src3, dst3, zeros2)

    b2p = jnp.pad(b2, (0, _D2 - b2.shape[0])).reshape(1, _D2)
    out16 = _fin_call(degc, agg2, y2, b2p)
    return out16[:, :8]
